# Initial kernel scaffold; baseline (speedup 1.0000x reference)
import jax
import jax.numpy as jnp
from jax.experimental import pallas as pl


def kernel(audio_ids, tables):
    raise NotImplementedError("write your pallas kernel here")


# speedup vs baseline: 5.8070x; 5.8070x over previous
import functools

import jax
import jax.numpy as jnp
from jax import lax
from jax.experimental import pallas as pl
from jax.experimental.pallas import tpu as pltpu
from jax.experimental.pallas import tpu_sc as plsc

NUM_CHANNEL = 8
VOCAB = 100000
HIDDEN = 128
MBS = 4
SEQ = 2048

NPOS = SEQ * MBS
LANES = 16

_info = plsc.get_sparse_core_info()
NC = _info.num_cores
NS = _info.num_subcores
NW = NC * NS
PPW = NPOS // NW
CHUNK = 16
NCHUNK = PPW // CHUNK

_mesh = plsc.VectorSubcoreMesh(core_axis_name="c", subcore_axis_name="s")


@functools.partial(
    pl.kernel,
    mesh=_mesh,
    out_type=jax.ShapeDtypeStruct((NPOS, HIDDEN), jnp.float32),
    scratch_types=[
        pltpu.VMEM((NUM_CHANNEL, NCHUNK, CHUNK), jnp.int32),
        pltpu.VMEM((NUM_CHANNEL, CHUNK, HIDDEN), jnp.float32),
        pltpu.VMEM((CHUNK, HIDDEN), jnp.float32),
        pltpu.SemaphoreType.DMA,
    ],
)
def _sc_embed(ids_hbm, tab_hbm, out_hbm, ids_v, gbuf, obuf, sem):
    wid = lax.axis_index("s") * NC + lax.axis_index("c")

    for c in range(NUM_CHANNEL):
        pltpu.sync_copy(ids_hbm.at[c, wid], ids_v.at[c])

    def chunk_body(k, carry):
        copies = []
        for c in range(NUM_CHANNEL):
            cp = pltpu.async_copy(
                tab_hbm.at[c].at[ids_v.at[c, k]], gbuf.at[c], sem
            )
            copies.append(cp)
        for cp in copies:
            cp.wait()

        def pos_body(p, carry2):
            for h in range(HIDDEN // LANES):
                sl = pl.ds(h * LANES, LANES)
                acc = gbuf[0, p, sl]
                for c in range(1, NUM_CHANNEL):
                    acc = acc + gbuf[c, p, sl]
                obuf[p, sl] = acc
            return carry2

        lax.fori_loop(0, CHUNK, pos_body, 0, unroll=False)

        base = wid * PPW + k * CHUNK
        pltpu.sync_copy(obuf, out_hbm.at[pl.ds(base, CHUNK)])
        return carry

    lax.fori_loop(0, NCHUNK, chunk_body, 0, unroll=False)


def kernel(audio_ids, tables):
    ids_t = jnp.transpose(audio_ids, (2, 1, 0)).reshape(
        NUM_CHANNEL, NW, NCHUNK, CHUNK
    )
    out = _sc_embed(ids_t, tables)
    return out.reshape(SEQ, MBS, HIDDEN)


# speedup vs baseline: 7.5198x; 1.2950x over previous
import functools

import jax
import jax.numpy as jnp
from jax import lax
from jax.experimental import pallas as pl
from jax.experimental.pallas import tpu as pltpu
from jax.experimental.pallas import tpu_sc as plsc

NUM_CHANNEL = 8
VOCAB = 100000
HIDDEN = 128
MBS = 4
SEQ = 2048

NPOS = SEQ * MBS
LANES = 16

_info = plsc.get_sparse_core_info()
NC = _info.num_cores
NS = _info.num_subcores
NW = NC * NS
PPW = NPOS // NW
CHUNK = 32
NCHUNK = PPW // CHUNK

_mesh = plsc.VectorSubcoreMesh(core_axis_name="c", subcore_axis_name="s")


@functools.partial(
    pl.kernel,
    mesh=_mesh,
    out_type=jax.ShapeDtypeStruct((NPOS, HIDDEN), jnp.float32),
    scratch_types=[
        pltpu.VMEM((NUM_CHANNEL, NCHUNK, CHUNK), jnp.int32),
        pltpu.VMEM((2, NUM_CHANNEL, CHUNK, HIDDEN), jnp.float32),
        pltpu.VMEM((2, CHUNK, HIDDEN), jnp.float32),
        pltpu.SemaphoreType.DMA,
        pltpu.SemaphoreType.DMA,
        pltpu.SemaphoreType.DMA,
        pltpu.SemaphoreType.DMA,
    ],
)
def _sc_embed(ids_hbm, tab_hbm, out_hbm, ids_v, gbuf, obuf, g0, g1, o0, o1):
    wid = lax.axis_index("s") * NC + lax.axis_index("c")
    gsem = (g0, g1)
    osem = (o0, o1)

    for c in range(NUM_CHANNEL):
        pltpu.sync_copy(ids_hbm.at[c, wid], ids_v.at[c])

    def fire(k, j):
        return [
            pltpu.async_copy(
                tab_hbm.at[c].at[ids_v.at[c, k]], gbuf.at[j, c], gsem[j]
            )
            for c in range(NUM_CHANNEL)
        ]

    gcopies = [fire(0, 0), None]
    scopies = [None, None]

    for k in range(NCHUNK):
        j = k % 2
        for cp in gcopies[j]:
            cp.wait()
        if k + 1 < NCHUNK:
            gcopies[(k + 1) % 2] = fire(k + 1, (k + 1) % 2)

        def pos_body(p, carry, _j=j):
            for h in range(HIDDEN // LANES):
                sl = pl.ds(h * LANES, LANES)
                acc = gbuf[_j, 0, p, sl]
                for c in range(1, NUM_CHANNEL):
                    acc = acc + gbuf[_j, c, p, sl]
                obuf[_j, p, sl] = acc
            return carry

        lax.fori_loop(0, CHUNK, pos_body, 0, unroll=False)

        if scopies[j] is not None:
            scopies[j].wait()
        base = wid * PPW + k * CHUNK
        scopies[j] = pltpu.async_copy(
            obuf.at[j], out_hbm.at[pl.ds(base, CHUNK)], osem[j]
        )

    for cp in scopies:
        if cp is not None:
            cp.wait()


def kernel(audio_ids, tables):
    ids_t = jnp.transpose(audio_ids, (2, 1, 0)).reshape(
        NUM_CHANNEL, NW, NCHUNK, CHUNK
    )
    out = _sc_embed(ids_t, tables)
    return out.reshape(SEQ, MBS, HIDDEN)
